# baseline (speedup 1.0000x reference)
import functools
import jax
import jax.numpy as jnp
from jax import lax
from jax.experimental import pallas as pl
from jax.experimental.pallas import tpu as pltpu
from jax.experimental.pallas import tpu_sc as plsc

B, N, L, D = 8, 9, 512, 768
S = N - 1
RPB_IN = N * L
RPB_OUT = S * L
KTC = 6
SCB = B - KTC
CHUNK = 4096

NC, NS = 2, 16
NW = NC * NS
WPB = NW // SCB
RPW = SCB * RPB_OUT // NW
CH = 64
NCH = RPW // CH

_mesh = plsc.VectorSubcoreMesh(core_axis_name="c", subcore_axis_name="s")


def _tc_kernel(valid_ref, reps_in, mask_in, reps_out, mask_out):
    b = pl.program_id(0)
    v = valid_ref[b]

    @pl.when(v != 0)
    def _copy():
        reps_out[...] = reps_in[...]
        mask_out[...] = mask_in[...]

    @pl.when(v == 0)
    def _zero():
        reps_out[...] = jnp.zeros_like(reps_out)
        mask_out[...] = jnp.zeros_like(mask_out)


@functools.partial(
    pl.kernel,
    out_type=[
        jax.ShapeDtypeStruct((SCB * RPB_OUT, D), jnp.float32),
        jax.ShapeDtypeStruct((SCB * S, 1, L), jnp.float32),
    ],
    mesh=_mesh,
    scratch_types=[
        pltpu.MemorySpace.VMEM((CH, D), jnp.float32),
        pltpu.MemorySpace.VMEM((CH, D), jnp.float32),
        pltpu.SemaphoreType.DMA,
        pltpu.SemaphoreType.DMA,
        pltpu.SemaphoreType.DMA,
        pltpu.SemaphoreType.DMA,
    ],
)
def _sc_select(reps_hbm, mask_hbm, reps_out, mask_out,
               buf0, buf1, sin0, sin1, sout0, sout1):
    c = lax.axis_index("c")
    s = lax.axis_index("s")
    wid = s * NC + c
    b = KTC + wid // WPB
    q = wid % WPB
    in_row = b * RPB_IN + L + q * RPW
    out_row = (b - KTC) * RPB_OUT + q * RPW

    bufs = (buf0, buf1)
    sins = (sin0, sin1)
    souts = (sout0, sout1)
    mk = pltpu.make_async_copy

    def in_cp(k):
        return mk(reps_hbm.at[pl.ds(in_row + k * CH, CH)], bufs[k % 2],
                  sins[k % 2])

    def out_cp(k):
        return mk(bufs[k % 2], reps_out.at[pl.ds(out_row + k * CH, CH)],
                  souts[k % 2])

    in_cp(0).start()
    for k in range(NCH):
        if k + 1 < NCH:
            if k >= 1:
                out_cp(k - 1).wait()
            in_cp(k + 1).start()
        in_cp(k).wait()
        out_cp(k).start()
    out_cp(NCH - 2).wait()
    out_cp(NCH - 1).wait()

    @pl.when(q == 0)
    def _copy_mask():
        pltpu.sync_copy(mask_hbm.at[pl.ds(b * N + 1, S)],
                        mask_out.at[pl.ds((b - KTC) * S, S)])


def kernel(token_reps, token_mask, valid_sentences):
    valid_i32 = valid_sentences.astype(jnp.int32)
    reps2d = token_reps.reshape(B * N * L, D)
    mask4 = token_mask.reshape(B, N, 1, L)
    mask3 = token_mask.reshape(B * N, 1, L)

    reps_tc, mask_tc = pl.pallas_call(
        _tc_kernel,
        grid=(KTC, 1),
        in_specs=[
            pl.BlockSpec(memory_space=pltpu.MemorySpace.SMEM),
            pl.BlockSpec(
                (pl.Element(CHUNK), pl.Element(D)),
                lambda b, c: (
                    pl.multiple_of(b * RPB_IN + L + c * CHUNK, 512),
                    0,
                ),
            ),
            pl.BlockSpec(
                (pl.Element(1), pl.Element(S), pl.Element(1), pl.Element(L)),
                lambda b, c: (b, 1, 0, 0),
            ),
        ],
        out_specs=[
            pl.BlockSpec((CHUNK, D), lambda b, c: (b + c, 0)),
            pl.BlockSpec((1, S, 1, L), lambda b, c: (b, 0, 0, 0)),
        ],
        out_shape=[
            jax.ShapeDtypeStruct((KTC * RPB_OUT, D), jnp.float32),
            jax.ShapeDtypeStruct((KTC, S, 1, L), jnp.float32),
        ],
    )(valid_i32, reps2d, mask4)

    reps_sc, mask_sc = _sc_select(reps2d, mask3)

    reps = jnp.concatenate([reps_tc, reps_sc], axis=0)
    mask = jnp.concatenate(
        [mask_tc.reshape(KTC * S, L), mask_sc.reshape(SCB * S, L)], axis=0)
    return reps.reshape(B, S, L, D), mask.reshape(B, S, L)


# speedup vs baseline: 2.1916x; 2.1916x over previous
import jax
import jax.numpy as jnp
from jax.experimental import pallas as pl
from jax.experimental.pallas import tpu as pltpu

B, N, L, D = 8, 9, 512, 768
S = N - 1
ROWS_PER_BATCH_IN = N * L
ROWS_PER_BATCH_OUT = S * L
CHUNK = 4096
CPB = ROWS_PER_BATCH_OUT // CHUNK


def _select_kernel(valid_ref, reps_in, mask_in, reps_out, mask_out):
    b = pl.program_id(0)
    v = valid_ref[b]

    @pl.when(v != 0)
    def _copy():
        reps_out[...] = reps_in[...]
        mask_out[...] = mask_in[...]

    @pl.when(v == 0)
    def _zero():
        reps_out[...] = jnp.zeros_like(reps_out)
        mask_out[...] = jnp.zeros_like(mask_out)


def kernel(token_reps, token_mask, valid_sentences):
    valid_i32 = valid_sentences.astype(jnp.int32)
    reps2d = token_reps.reshape(B * N * L, D)
    mask4 = token_mask.reshape(B, N, 1, L)

    reps_out, mask_out = pl.pallas_call(
        _select_kernel,
        grid=(B, CPB),
        in_specs=[
            pl.BlockSpec(memory_space=pltpu.MemorySpace.SMEM),
            pl.BlockSpec(
                (pl.Element(CHUNK), pl.Element(D)),
                lambda b, c: (
                    pl.multiple_of(b * ROWS_PER_BATCH_IN + L + c * CHUNK, 512),
                    0,
                ),
            ),
            pl.BlockSpec(
                (pl.Element(1), pl.Element(S), pl.Element(1), pl.Element(L)),
                lambda b, c: (b, 1, 0, 0),
            ),
        ],
        out_specs=[
            pl.BlockSpec((CHUNK, D), lambda b, c: (b * CPB + c, 0)),
            pl.BlockSpec((1, S, 1, L), lambda b, c: (b, 0, 0, 0)),
        ],
        out_shape=[
            jax.ShapeDtypeStruct((B * S * L, D), jnp.float32),
            jax.ShapeDtypeStruct((B, S, 1, L), jnp.float32),
        ],
    )(valid_i32, reps2d, mask4)

    return reps_out.reshape(B, S, L, D), mask_out.reshape(B, S, L)
